# baseline (speedup 1.0000x reference)
import dataclasses
import functools

import jax
import jax.numpy as jnp
from jax import lax
from jax.experimental import pallas as pl
from jax.experimental.pallas import tpu as pltpu
from jax.experimental.pallas import tpu_sc as plsc

VOCAB = 1000000
DIM = 64
L = 200
B = 4096
HID = 1024
LABELS = 1000
EPS = 1e-5

NC = 2
NS = 16
LANES = 16
NW = NC * NS
BPW = B // NW
W2DIM = 2 * DIM
NRC = BPW // LANES


def _sc_pool_sum(idx, table2):
    mesh = plsc.VectorSubcoreMesh(core_axis_name="c", subcore_axis_name="s")
    cp = pltpu.CompilerParams()
    if "needs_layout_passes" in pltpu.CompilerParams.__dataclass_fields__:
        cp = dataclasses.replace(cp, needs_layout_passes=False)

    @functools.partial(
        pl.kernel,
        mesh=mesh,
        compiler_params=cp,
        out_type=jax.ShapeDtypeStruct((DIM, B), jnp.float32),
        scratch_types=[
            pltpu.VMEM((L, BPW), jnp.int32),
            pltpu.VMEM((L, BPW), jnp.int32),
            pltpu.VMEM((BPW, W2DIM), jnp.float32),
            pltpu.VMEM((BPW, W2DIM), jnp.float32),
            pltpu.VMEM((DIM, BPW), jnp.float32),
            pltpu.SemaphoreType.DMA,
            pltpu.SemaphoreType.DMA,
        ],
    )
    def sc_kernel(idx_hbm, table_hbm, out_hbm, idx_v, idxs_v, g0, g1, accT,
                  sem0, sem1):
        wid = lax.axis_index("s") * NC + lax.axis_index("c")
        base = wid * BPW

        pltpu.sync_copy(idx_hbm.at[:, pl.ds(base, BPW)], idx_v)

        @pl.loop(0, L, step=4)
        def _(l):
            for ll in range(4):
                for c in range(NRC):
                    sl = (l + ll, pl.ds(LANES * c, LANES))
                    idxs_v[sl] = lax.shift_right_logical(idx_v[sl], 1)

        def issue(l, gbuf, sem):
            pltpu.async_copy(table_hbm.at[idxs_v.at[l]], gbuf, sem)

        def drain(gbuf, sem):
            pltpu.make_async_copy(table_hbm.at[idxs_v.at[0]], gbuf, sem).wait()

        iota = lax.iota(jnp.int32, LANES)

        def accum(gbuf, l, first):
            for rc in range(NRC):
                idxch = idx_v[l, pl.ds(rc * LANES, LANES)]
                colbase = (idxch & 1) * DIM
                rowv = iota + (rc * LANES)

                @pl.loop(0, DIM, step=8)
                def _(c0):
                    for cc in range(8):
                        colv = colbase + (c0 + cc)
                        vals = plsc.load_gather(gbuf, [rowv, colv])
                        dsl = (c0 + cc, pl.ds(rc * LANES, LANES))
                        if first:
                            accT[dsl] = vals
                        else:
                            accT[dsl] = accT[dsl] + vals

        issue(0, g0, sem0)
        issue(1, g1, sem1)

        drain(g0, sem0)
        accum(g0, 0, first=True)
        issue(2, g0, sem0)
        drain(g1, sem1)
        accum(g1, 1, first=False)
        issue(3, g1, sem1)

        @pl.loop(0, (L - 4) // 2)
        def _(k):
            drain(g0, sem0)
            accum(g0, 2 * k + 2, first=False)
            issue(2 * k + 4, g0, sem0)
            drain(g1, sem1)
            accum(g1, 2 * k + 3, first=False)
            issue(2 * k + 5, g1, sem1)

        drain(g0, sem0)
        accum(g0, L - 2, first=False)
        drain(g1, sem1)
        accum(g1, L - 1, first=False)

        pltpu.sync_copy(accT, out_hbm.at[:, pl.ds(base, BPW)])

    return sc_kernel(idx, table2)


def _stats_body(cs_ref, w1_ref, b1_ref, gamma_ref, beta_ref, s2_ref, t2_ref):
    cs = cs_ref[...]
    w1 = w1_ref[...]
    mean_c = jnp.sum(cs, axis=1, keepdims=True) * (1.0 / (B * L))
    m2 = lax.dot_general(
        cs, cs, (((1,), (1,)), ((), ())),
        preferred_element_type=jnp.float32,
        precision=lax.Precision.HIGHEST,
    ) * (1.0 / (B * L * L))
    outer = lax.dot_general(
        mean_c, mean_c, (((1,), (1,)), ((), ())),
        preferred_element_type=jnp.float32,
        precision=lax.Precision.HIGHEST,
    )
    cov = m2 - outer
    t = lax.dot_general(
        cov, w1, (((1,), (0,)), ((), ())),
        preferred_element_type=jnp.float32,
        precision=lax.Precision.HIGHEST,
    )
    var = jnp.sum(w1 * t, axis=0, keepdims=True)
    mean_h = lax.dot_general(
        mean_c, w1, (((0,), (0,)), ((), ())),
        preferred_element_type=jnp.float32,
        precision=lax.Precision.HIGHEST,
    ) + b1_ref[...]
    s = gamma_ref[...] * lax.rsqrt(var + EPS)
    s2_ref[...] = s * (1.0 / L)
    t2_ref[...] = b1_ref[...] * s + beta_ref[...] - mean_h * s


def _main_body(cs_ref, w1_ref, w2_ref, b2_ref, s2_ref, t2_ref, out_ref):
    mm = lax.dot_general(
        cs_ref[...], w1_ref[...], (((0,), (0,)), ((), ())),
        preferred_element_type=jnp.float32,
    )
    hn = jnp.maximum(mm * s2_ref[...] + t2_ref[...], 0.0)
    out_ref[...] = (
        jnp.dot(hn, w2_ref[...], preferred_element_type=jnp.float32)
        + b2_ref[...]
    )


BB = 512
NB = B // BB


def kernel(input, W_emb, W1, b1, gamma, beta, W2, b2):
    table2 = W_emb.reshape(VOCAB // 2, W2DIM)
    csumT = _sc_pool_sum(input, table2)

    b1r = b1.reshape(1, HID)
    gr = gamma.reshape(1, HID)
    ber = beta.reshape(1, HID)
    b2r = b2.reshape(1, LABELS)

    s2, t2 = pl.pallas_call(
        _stats_body,
        out_shape=[
            jax.ShapeDtypeStruct((1, HID), jnp.float32),
            jax.ShapeDtypeStruct((1, HID), jnp.float32),
        ],
    )(csumT, W1, b1r, gr, ber)

    out = pl.pallas_call(
        _main_body,
        grid=(NB,),
        in_specs=[
            pl.BlockSpec((DIM, BB), lambda i: (0, i)),
            pl.BlockSpec((DIM, HID), lambda i: (0, 0)),
            pl.BlockSpec((HID, LABELS), lambda i: (0, 0)),
            pl.BlockSpec((1, LABELS), lambda i: (0, 0)),
            pl.BlockSpec((1, HID), lambda i: (0, 0)),
            pl.BlockSpec((1, HID), lambda i: (0, 0)),
        ],
        out_specs=pl.BlockSpec((BB, LABELS), lambda i: (i, 0)),
        out_shape=jax.ShapeDtypeStruct((B, LABELS), jnp.float32),
    )(csumT, W1, W2, b2r, s2, t2)

    return out


# speedup vs baseline: 2.5742x; 2.5742x over previous
import functools

import jax
import jax.numpy as jnp
from jax import lax
from jax.experimental import pallas as pl
from jax.experimental.pallas import tpu as pltpu
from jax.experimental.pallas import tpu_sc as plsc

VOCAB = 1000000
DIM = 64
L = 200
B = 4096
HID = 1024
LABELS = 1000
EPS = 1e-5

NC = 2
NS = 16
LANES = 16
NW = NC * NS
BPW = B // NW
ROW_UNROLL = 8


PDIM = 128


def _sc_pool_sum(idx, table):
    mesh = plsc.VectorSubcoreMesh(core_axis_name="c", subcore_axis_name="s")

    @functools.partial(
        pl.kernel,
        mesh=mesh,
        out_type=jax.ShapeDtypeStruct((B, DIM), jnp.float32),
        scratch_types=[
            pltpu.VMEM((L, BPW), jnp.int32),
            pltpu.VMEM((BPW, PDIM), jnp.float32),
            pltpu.VMEM((BPW, PDIM), jnp.float32),
            pltpu.VMEM((BPW, DIM), jnp.float32),
            pltpu.SemaphoreType.DMA,
            pltpu.SemaphoreType.DMA,
        ],
    )
    def sc_kernel(idx_hbm, table_hbm, out_hbm, idx_v, g0, g1, acc, sem0, sem1):
        wid = lax.axis_index("s") * NC + lax.axis_index("c")
        base = wid * BPW

        pltpu.sync_copy(idx_hbm.at[:, pl.ds(base, BPW)], idx_v)

        def issue(l, gbuf, sem):
            pltpu.async_copy(table_hbm.at[idx_v.at[l]], gbuf, sem)

        def drain(gbuf, sem):
            pltpu.make_async_copy(table_hbm.at[idx_v.at[0]], gbuf, sem).wait()

        def accum(gbuf, first):
            @pl.loop(0, BPW, step=ROW_UNROLL)
            def _(r):
                for rr in range(ROW_UNROLL):
                    for c in range(DIM // LANES):
                        sl = (pl.ds(r + rr, 1), pl.ds(LANES * c, LANES))
                        if first:
                            acc[sl] = gbuf[sl]
                        else:
                            acc[sl] = acc[sl] + gbuf[sl]

        issue(0, g0, sem0)
        issue(1, g1, sem1)

        drain(g0, sem0)
        accum(g0, first=True)
        issue(2, g0, sem0)
        drain(g1, sem1)
        accum(g1, first=False)
        issue(3, g1, sem1)

        @pl.loop(0, (L - 4) // 2)
        def _(k):
            drain(g0, sem0)
            accum(g0, first=False)
            issue(2 * k + 4, g0, sem0)
            drain(g1, sem1)
            accum(g1, first=False)
            issue(2 * k + 5, g1, sem1)

        drain(g0, sem0)
        accum(g0, first=False)
        drain(g1, sem1)
        accum(g1, first=False)

        pltpu.sync_copy(acc, out_hbm.at[pl.ds(base, BPW)])

    return sc_kernel(idx, table)


def _stats_body(cs_ref, w1_ref, b1_ref, gamma_ref, beta_ref, s2_ref, t2_ref):
    cs = cs_ref[...]
    w1 = w1_ref[...]
    mean_c = jnp.sum(cs, axis=0, keepdims=True) * (1.0 / (B * L))
    m2 = lax.dot_general(
        cs, cs, (((0,), (0,)), ((), ())),
        preferred_element_type=jnp.float32,
        precision=lax.Precision.HIGHEST,
    ) * (1.0 / (B * L * L))
    outer = lax.dot_general(
        mean_c, mean_c, (((0,), (0,)), ((), ())),
        preferred_element_type=jnp.float32,
        precision=lax.Precision.HIGHEST,
    )
    cov = m2 - outer
    t = lax.dot_general(
        cov, w1, (((1,), (0,)), ((), ())),
        preferred_element_type=jnp.float32,
        precision=lax.Precision.HIGHEST,
    )
    var = jnp.sum(w1 * t, axis=0, keepdims=True)
    mean_h = lax.dot_general(
        mean_c, w1, (((1,), (0,)), ((), ())),
        preferred_element_type=jnp.float32,
        precision=lax.Precision.HIGHEST,
    ) + b1_ref[...]
    s = gamma_ref[...] * lax.rsqrt(var + EPS)
    s2_ref[...] = s * (1.0 / L)
    t2_ref[...] = b1_ref[...] * s + beta_ref[...] - mean_h * s


def _main_body(cs_ref, w1_ref, w2_ref, b2_ref, s2_ref, t2_ref, out_ref):
    mm = jnp.dot(cs_ref[...], w1_ref[...], preferred_element_type=jnp.float32)
    hn = jnp.maximum(mm * s2_ref[...] + t2_ref[...], 0.0)
    out_ref[...] = (
        jnp.dot(hn, w2_ref[...], preferred_element_type=jnp.float32)
        + b2_ref[...]
    )


BB = 512
NB = B // BB


def kernel(input, W_emb, W1, b1, gamma, beta, W2, b2):
    table_p = jnp.pad(W_emb, ((0, 0), (0, PDIM - DIM)))
    csum = _sc_pool_sum(input, table_p)

    b1r = b1.reshape(1, HID)
    gr = gamma.reshape(1, HID)
    ber = beta.reshape(1, HID)
    b2r = b2.reshape(1, LABELS)

    s2, t2 = pl.pallas_call(
        _stats_body,
        out_shape=[
            jax.ShapeDtypeStruct((1, HID), jnp.float32),
            jax.ShapeDtypeStruct((1, HID), jnp.float32),
        ],
    )(csum, W1, b1r, gr, ber)

    out = pl.pallas_call(
        _main_body,
        grid=(NB,),
        in_specs=[
            pl.BlockSpec((BB, DIM), lambda i: (i, 0)),
            pl.BlockSpec((DIM, HID), lambda i: (0, 0)),
            pl.BlockSpec((HID, LABELS), lambda i: (0, 0)),
            pl.BlockSpec((1, LABELS), lambda i: (0, 0)),
            pl.BlockSpec((1, HID), lambda i: (0, 0)),
            pl.BlockSpec((1, HID), lambda i: (0, 0)),
        ],
        out_specs=pl.BlockSpec((BB, LABELS), lambda i: (i, 0)),
        out_shape=jax.ShapeDtypeStruct((B, LABELS), jnp.float32),
    )(csum, W1, W2, b2r, s2, t2)

    return out


# speedup vs baseline: 2.7627x; 1.0732x over previous
import functools

import jax
import jax.numpy as jnp
from jax import lax
from jax.experimental import pallas as pl
from jax.experimental.pallas import tpu as pltpu
from jax.experimental.pallas import tpu_sc as plsc

VOCAB = 1000000
DIM = 64
L = 200
B = 4096
HID = 1024
LABELS = 1000
EPS = 1e-5

NC = 2
NS = 16
LANES = 16
NW = NC * NS
BPW = B // NW
ROW_UNROLL = 8


PDIM = 128


def _sc_pool_sum(idx, table):
    mesh = plsc.VectorSubcoreMesh(core_axis_name="c", subcore_axis_name="s")

    @functools.partial(
        pl.kernel,
        mesh=mesh,
        out_type=jax.ShapeDtypeStruct((B, DIM), jnp.float32),
        scratch_types=[
            pltpu.VMEM((L, BPW), jnp.int32),
            pltpu.VMEM((BPW, PDIM), jnp.float32),
            pltpu.VMEM((BPW, PDIM), jnp.float32),
            pltpu.VMEM((BPW, DIM), jnp.float32),
            pltpu.SemaphoreType.DMA,
            pltpu.SemaphoreType.DMA,
        ],
    )
    def sc_kernel(idx_hbm, table_hbm, out_hbm, idx_v, g0, g1, acc, sem0, sem1):
        wid = lax.axis_index("s") * NC + lax.axis_index("c")
        base = wid * BPW

        pltpu.sync_copy(idx_hbm.at[:, pl.ds(base, BPW)], idx_v)

        def issue(l, gbuf, sem):
            pltpu.async_copy(table_hbm.at[idx_v.at[l]], gbuf, sem)

        def drain(gbuf, sem):
            pltpu.make_async_copy(table_hbm.at[idx_v.at[0]], gbuf, sem).wait()

        def accum(gbuf, first):
            @pl.loop(0, BPW, step=ROW_UNROLL)
            def _(r):
                for rr in range(ROW_UNROLL):
                    for c in range(DIM // LANES):
                        sl = (pl.ds(r + rr, 1), pl.ds(LANES * c, LANES))
                        if first:
                            acc[sl] = gbuf[sl]
                        else:
                            acc[sl] = acc[sl] + gbuf[sl]

        issue(0, g0, sem0)
        issue(1, g1, sem1)

        drain(g0, sem0)
        accum(g0, first=True)
        issue(2, g0, sem0)
        drain(g1, sem1)
        accum(g1, first=False)
        issue(3, g1, sem1)

        @pl.loop(0, (L - 4) // 2)
        def _(k):
            drain(g0, sem0)
            accum(g0, first=False)
            issue(2 * k + 4, g0, sem0)
            drain(g1, sem1)
            accum(g1, first=False)
            issue(2 * k + 5, g1, sem1)

        drain(g0, sem0)
        accum(g0, first=False)
        drain(g1, sem1)
        accum(g1, first=False)

        pltpu.sync_copy(acc, out_hbm.at[pl.ds(base, BPW)])

    return sc_kernel(idx, table)


VCHUNK = 4096


def _relayout_body(tt_ref, out_ref):
    tt = tt_ref[...]
    eye = (lax.broadcasted_iota(jnp.int32, (DIM, DIM), 0)
           == lax.broadcasted_iota(jnp.int32, (DIM, DIM), 1)
           ).astype(jnp.float32)
    bt = lax.dot_general(
        tt, eye, (((0,), (0,)), ((), ())),
        preferred_element_type=jnp.float32,
        precision=lax.Precision.HIGHEST,
    )
    out_ref[:, 0:DIM] = bt


def _relayout_table(tableT):
    return pl.pallas_call(
        _relayout_body,
        grid=(VOCAB // VCHUNK,),
        in_specs=[pl.BlockSpec((DIM, VCHUNK), lambda i: (0, i))],
        out_specs=pl.BlockSpec((VCHUNK, PDIM), lambda i: (i, 0)),
        out_shape=jax.ShapeDtypeStruct((VOCAB, PDIM), jnp.float32),
    )(tableT)


def _stats_body(cs_ref, w1_ref, b1_ref, gamma_ref, beta_ref, s2_ref, t2_ref):
    cs = cs_ref[...]
    w1 = w1_ref[...]
    mean_c = jnp.sum(cs, axis=0, keepdims=True) * (1.0 / (B * L))
    m2 = lax.dot_general(
        cs, cs, (((0,), (0,)), ((), ())),
        preferred_element_type=jnp.float32,
        precision=lax.Precision.HIGHEST,
    ) * (1.0 / (B * L * L))
    outer = lax.dot_general(
        mean_c, mean_c, (((0,), (0,)), ((), ())),
        preferred_element_type=jnp.float32,
        precision=lax.Precision.HIGHEST,
    )
    cov = m2 - outer
    t = lax.dot_general(
        cov, w1, (((1,), (0,)), ((), ())),
        preferred_element_type=jnp.float32,
        precision=lax.Precision.HIGHEST,
    )
    var = jnp.sum(w1 * t, axis=0, keepdims=True)
    mean_h = lax.dot_general(
        mean_c, w1, (((1,), (0,)), ((), ())),
        preferred_element_type=jnp.float32,
        precision=lax.Precision.HIGHEST,
    ) + b1_ref[...]
    s = gamma_ref[...] * lax.rsqrt(var + EPS)
    s2_ref[...] = s * (1.0 / L)
    t2_ref[...] = b1_ref[...] * s + beta_ref[...] - mean_h * s


def _main_body(cs_ref, w1_ref, w2_ref, b2_ref, s2_ref, t2_ref, out_ref):
    mm = jnp.dot(cs_ref[...], w1_ref[...], preferred_element_type=jnp.float32)
    hn = jnp.maximum(mm * s2_ref[...] + t2_ref[...], 0.0)
    out_ref[...] = (
        jnp.dot(hn, w2_ref[...], preferred_element_type=jnp.float32)
        + b2_ref[...]
    )


BB = 512
NB = B // BB


def kernel(input, W_emb, W1, b1, gamma, beta, W2, b2):
    table_p = _relayout_table(W_emb.T)
    csum = _sc_pool_sum(input, table_p)

    b1r = b1.reshape(1, HID)
    gr = gamma.reshape(1, HID)
    ber = beta.reshape(1, HID)
    b2r = b2.reshape(1, LABELS)

    s2, t2 = pl.pallas_call(
        _stats_body,
        out_shape=[
            jax.ShapeDtypeStruct((1, HID), jnp.float32),
            jax.ShapeDtypeStruct((1, HID), jnp.float32),
        ],
    )(csum, W1, b1r, gr, ber)

    out = pl.pallas_call(
        _main_body,
        grid=(NB,),
        in_specs=[
            pl.BlockSpec((BB, DIM), lambda i: (i, 0)),
            pl.BlockSpec((DIM, HID), lambda i: (0, 0)),
            pl.BlockSpec((HID, LABELS), lambda i: (0, 0)),
            pl.BlockSpec((1, LABELS), lambda i: (0, 0)),
            pl.BlockSpec((1, HID), lambda i: (0, 0)),
            pl.BlockSpec((1, HID), lambda i: (0, 0)),
        ],
        out_specs=pl.BlockSpec((BB, LABELS), lambda i: (i, 0)),
        out_shape=jax.ShapeDtypeStruct((B, LABELS), jnp.float32),
    )(csum, W1, W2, b2r, s2, t2)

    return out
